# baseline (device time: 334816 ns/iter reference)
import jax
import jax.numpy as jnp
from jax import lax
from jax.experimental import pallas as pl
from jax.experimental.pallas import tpu as pltpu

_MESH = pl.DeviceIdType.MESH


def _headmajor(K2, V2, s, h, d):

    def body(k_ref, v_ref, kh_ref, vh_ref):
        kh_ref[0] = k_ref[...]
        vh_ref[0] = v_ref[...]

    col = pl.BlockSpec((s, d), lambda i: (0, i))
    head = pl.BlockSpec((1, s, d), lambda i: (i, 0, 0))
    return pl.pallas_call(
        body,
        grid=(h,),
        out_shape=[
            jax.ShapeDtypeStruct((h, s, d), K2.dtype),
            jax.ShapeDtypeStruct((h, s, d), V2.dtype),
        ],
        in_specs=[col, col],
        out_specs=[head, head],
    )(K2, V2)


def _fused(Q2, K2, V2, Kh, Vh, s, h, d):
    scale = d ** -0.5
    dn_qk = (((1,), (1,)), ((), ()))
    dn_pv = (((1,), (0,)), ((), ()))

    def body(q_ref, k_ref, v_ref, kh_ref, vh_ref, out_ref,
             kvr, recv_sems, xsend_sems, ysend_sems):
        pid = pl.program_id(0)
        my_x = lax.axis_index("x")
        my_y = lax.axis_index("y")
        xpeer = (1 - my_x, my_y)
        ypeer = (my_x, 1 - my_y)

        @pl.when(pid == 0)
        def _():
            barrier_sem = pltpu.get_barrier_semaphore()
            for p in (xpeer, ypeer):
                pl.semaphore_signal(barrier_sem, inc=1, device_id=p,
                                    device_id_type=_MESH)
            pl.semaphore_wait(barrier_sem, 2)
            for j in range(h // 2):
                hh = 2 * j + my_y
                for kv, src in ((0, kh_ref), (1, vh_ref)):
                    pltpu.make_async_remote_copy(
                        src_ref=src.at[hh], dst_ref=kvr.at[kv, hh],
                        send_sem=xsend_sems.at[kv, hh],
                        recv_sem=recv_sems.at[kv, hh],
                        device_id=xpeer, device_id_type=_MESH,
                    ).start()

        for kv in (0, 1):
            pltpu.make_async_remote_copy(
                src_ref=kvr.at[kv, pid], dst_ref=kvr.at[kv, pid],
                send_sem=xsend_sems.at[kv, pid],
                recv_sem=recv_sems.at[kv, pid],
                device_id=xpeer, device_id_type=_MESH,
            ).wait_recv()

        @pl.when(pid % 2 == my_y)
        def _():
            for kv in (0, 1):
                pltpu.make_async_remote_copy(
                    src_ref=kvr.at[kv, pid], dst_ref=kvr.at[kv, pid],
                    send_sem=ysend_sems.at[kv, pid],
                    recv_sem=recv_sems.at[kv, pid],
                    device_id=ypeer, device_id_type=_MESH,
                ).start()

        q = q_ref[...]
        s1 = lax.dot_general(q, k_ref[...], dn_qk,
                             preferred_element_type=jnp.float32) * scale
        s2 = lax.dot_general(q, kvr[0, pid], dn_qk,
                             preferred_element_type=jnp.float32) * scale
        p1 = jnp.exp(s1)
        p2 = jnp.exp(s2)
        denom = (jnp.sum(p1, axis=1, keepdims=True)
                 + jnp.sum(p2, axis=1, keepdims=True))
        o = lax.dot_general(p1, v_ref[...], dn_pv,
                            preferred_element_type=jnp.float32)
        o = o + lax.dot_general(p2, kvr[1, pid], dn_pv,
                                preferred_element_type=jnp.float32)
        out_ref[...] = o / denom

        @pl.when(pid == h - 1)
        def _():
            for j in range(h // 2):
                hh = 2 * j + my_y
                for kv in (0, 1):
                    for sems, peer in ((xsend_sems, xpeer),
                                       (ysend_sems, ypeer)):
                        pltpu.make_async_remote_copy(
                            src_ref=kvr.at[kv, hh], dst_ref=kvr.at[kv, hh],
                            send_sem=sems.at[kv, hh],
                            recv_sem=recv_sems.at[kv, hh],
                            device_id=peer, device_id_type=_MESH,
                        ).wait_send()

    col = pl.BlockSpec((s, d), lambda i: (0, i))
    return pl.pallas_call(
        body,
        grid=(h,),
        out_shape=jax.ShapeDtypeStruct((s, h * d), jnp.float32),
        in_specs=[
            col, col, col,
            pl.BlockSpec(memory_space=pl.ANY),
            pl.BlockSpec(memory_space=pl.ANY),
        ],
        out_specs=col,
        scratch_shapes=[
            pltpu.VMEM((2, h, s, d), jnp.float32),
            pltpu.SemaphoreType.DMA((2, h)),
            pltpu.SemaphoreType.DMA((2, h)),
            pltpu.SemaphoreType.DMA((2, h)),
        ],
        compiler_params=pltpu.CompilerParams(collective_id=0),
    )(Q2, K2, V2, Kh, Vh)


def kernel(Q, K, V):
    b, s, h, d = Q.shape

    Q2 = Q.reshape(s, h * d)
    K2 = K.reshape(s, h * d)
    V2 = V.reshape(s, h * d)

    Kh, Vh = _headmajor(K2, V2, s, h, d)
    out2 = _fused(Q2, K2, V2, Kh, Vh, s, h, d)
    return out2.reshape(b, s, h, d)


# device time: 176460 ns/iter; 1.8974x vs baseline; 1.8974x over previous
import jax
import jax.numpy as jnp
from jax import lax
from jax.experimental import pallas as pl
from jax.experimental.pallas import tpu as pltpu

_MESH = pl.DeviceIdType.MESH


def _headmajor(K2, V2, s, h, d):

    def body(k_ref, v_ref, kh_ref, vh_ref):
        kh_ref[0] = k_ref[...]
        vh_ref[0] = v_ref[...]

    col = pl.BlockSpec((s, d), lambda i: (0, i))
    head = pl.BlockSpec((1, s, d), lambda i: (i, 0, 0))
    return pl.pallas_call(
        body,
        grid=(h,),
        out_shape=[
            jax.ShapeDtypeStruct((h, s, d), K2.dtype),
            jax.ShapeDtypeStruct((h, s, d), V2.dtype),
        ],
        in_specs=[col, col],
        out_specs=[head, head],
    )(K2, V2)


def _fused(Q2, K2, V2, Kh, Vh, s, h, d):
    scale = d ** -0.5
    dn_qk = (((1,), (1,)), ((), ()))
    dn_pv = (((1,), (0,)), ((), ()))

    def body(q_ref, k_ref, v_ref, kh_ref, vh_ref, out_ref,
             kvr, recv_sems, xsend_sems, ysend_sems):
        pid = pl.program_id(0)
        my_x = lax.axis_index("x")
        my_y = lax.axis_index("y")
        xpeer = (1 - my_x, my_y)
        ypeer = (my_x, 1 - my_y)

        @pl.when(pid == 0)
        def _():
            barrier_sem = pltpu.get_barrier_semaphore()
            for p in (xpeer, ypeer):
                pl.semaphore_signal(barrier_sem, inc=1, device_id=p,
                                    device_id_type=_MESH)
            pl.semaphore_wait(barrier_sem, 2)
            for j in range(h // 2):
                hh = 2 * j + my_y
                for kv, src in ((0, kh_ref), (1, vh_ref)):
                    pltpu.make_async_remote_copy(
                        src_ref=src.at[hh], dst_ref=kvr.at[kv, hh],
                        send_sem=xsend_sems.at[kv, hh],
                        recv_sem=recv_sems.at[kv, hh],
                        device_id=xpeer, device_id_type=_MESH,
                    ).start()

        @pl.when(pid < h // 2)
        def _():
            hh = 2 * pid + my_y
            for kv in (0, 1):
                pltpu.make_async_remote_copy(
                    src_ref=kvr.at[kv, hh], dst_ref=kvr.at[kv, hh],
                    send_sem=xsend_sems.at[kv, hh],
                    recv_sem=recv_sems.at[kv, hh],
                    device_id=xpeer, device_id_type=_MESH,
                ).wait_recv()
                pltpu.make_async_remote_copy(
                    src_ref=kvr.at[kv, hh], dst_ref=kvr.at[kv, hh],
                    send_sem=ysend_sems.at[kv, hh],
                    recv_sem=recv_sems.at[kv, hh],
                    device_id=ypeer, device_id_type=_MESH,
                ).start()

        @pl.when(pid % 2 != my_y)
        def _():
            for kv in (0, 1):
                pltpu.make_async_remote_copy(
                    src_ref=kvr.at[kv, pid], dst_ref=kvr.at[kv, pid],
                    send_sem=xsend_sems.at[kv, pid],
                    recv_sem=recv_sems.at[kv, pid],
                    device_id=xpeer, device_id_type=_MESH,
                ).wait_recv()

        q = q_ref[...]
        s1 = lax.dot_general(q, k_ref[...], dn_qk,
                             preferred_element_type=jnp.float32) * scale
        s2 = lax.dot_general(q, kvr[0, pid], dn_qk,
                             preferred_element_type=jnp.float32) * scale
        p1 = jnp.exp(s1)
        p2 = jnp.exp(s2)
        denom = (jnp.sum(p1, axis=1, keepdims=True)
                 + jnp.sum(p2, axis=1, keepdims=True))
        o = lax.dot_general(p1, v_ref[...], dn_pv,
                            preferred_element_type=jnp.float32)
        o = o + lax.dot_general(p2, kvr[1, pid], dn_pv,
                                preferred_element_type=jnp.float32)
        out_ref[...] = o / denom

        @pl.when(pid == h - 1)
        def _():
            for j in range(h // 2):
                hh = 2 * j + my_y
                for kv in (0, 1):
                    for sems, peer in ((xsend_sems, xpeer),
                                       (ysend_sems, ypeer)):
                        pltpu.make_async_remote_copy(
                            src_ref=kvr.at[kv, hh], dst_ref=kvr.at[kv, hh],
                            send_sem=sems.at[kv, hh],
                            recv_sem=recv_sems.at[kv, hh],
                            device_id=peer, device_id_type=_MESH,
                        ).wait_send()

    col = pl.BlockSpec((s, d), lambda i: (0, i))
    return pl.pallas_call(
        body,
        grid=(h,),
        out_shape=jax.ShapeDtypeStruct((s, h * d), jnp.float32),
        in_specs=[
            col, col, col,
            pl.BlockSpec(memory_space=pl.ANY),
            pl.BlockSpec(memory_space=pl.ANY),
        ],
        out_specs=col,
        scratch_shapes=[
            pltpu.VMEM((2, h, s, d), jnp.float32),
            pltpu.SemaphoreType.DMA((2, h)),
            pltpu.SemaphoreType.DMA((2, h)),
            pltpu.SemaphoreType.DMA((2, h)),
        ],
        compiler_params=pltpu.CompilerParams(collective_id=0),
    )(Q2, K2, V2, Kh, Vh)


def kernel(Q, K, V):
    b, s, h, d = Q.shape

    Q2 = Q.reshape(s, h * d)
    K2 = K.reshape(s, h * d)
    V2 = V.reshape(s, h * d)

    Kh, Vh = _headmajor(K2, V2, s, h, d)
    out2 = _fused(Q2, K2, V2, Kh, Vh, s, h, d)
    return out2.reshape(b, s, h, d)


# device time: 158853 ns/iter; 2.1077x vs baseline; 1.1108x over previous
import jax
import jax.numpy as jnp
from jax import lax
from jax.experimental import pallas as pl
from jax.experimental.pallas import tpu as pltpu

_MESH = pl.DeviceIdType.MESH


def _headmajor(K2, V2, s, h, d):

    def body(k_ref, v_ref, kh_ref, vh_ref):
        kh_ref[0] = k_ref[...]
        vh_ref[0] = v_ref[...]

    col = pl.BlockSpec((s, d), lambda i: (0, i))
    head = pl.BlockSpec((1, s, d), lambda i: (i, 0, 0))
    return pl.pallas_call(
        body,
        grid=(h,),
        out_shape=[
            jax.ShapeDtypeStruct((h, s, d), K2.dtype),
            jax.ShapeDtypeStruct((h, s, d), V2.dtype),
        ],
        in_specs=[col, col],
        out_specs=[head, head],
    )(K2, V2)


def _fused(Q2, K2, V2, s, h, d):
    scale = d ** -0.5
    dn_qk = (((1,), (1,)), ((), ()))
    dn_pv = (((1,), (0,)), ((), ()))

    def body(q_ref, k_ref, v_ref, kh_ref, vh_ref, out_ref,
             kvr, recv_sems, xsend_sems, ysend_sems):
        pid = pl.program_id(0)
        my_x = lax.axis_index("x")
        my_y = lax.axis_index("y")
        xpeer = (1 - my_x, my_y)
        ypeer = (my_x, 1 - my_y)

        @pl.when(pid == 0)
        def _():
            barrier_sem = pltpu.get_barrier_semaphore()
            for p in (xpeer, ypeer):
                pl.semaphore_signal(barrier_sem, inc=1, device_id=p,
                                    device_id_type=_MESH)
            pl.semaphore_wait(barrier_sem, 2)
            for j in range(h // 2):
                hh = 2 * j + my_y
                for kv, src in ((0, kh_ref), (1, vh_ref)):
                    pltpu.make_async_remote_copy(
                        src_ref=src.at[:, pl.ds(hh * d, d)], dst_ref=kvr.at[kv, hh],
                        send_sem=xsend_sems.at[kv, hh],
                        recv_sem=recv_sems.at[kv, hh],
                        device_id=xpeer, device_id_type=_MESH,
                    ).start()

        @pl.when(pid < h // 2)
        def _():
            hh = 2 * pid + my_y
            for kv in (0, 1):
                pltpu.make_async_remote_copy(
                    src_ref=kvr.at[kv, hh], dst_ref=kvr.at[kv, hh],
                    send_sem=xsend_sems.at[kv, hh],
                    recv_sem=recv_sems.at[kv, hh],
                    device_id=xpeer, device_id_type=_MESH,
                ).wait_recv()
                pltpu.make_async_remote_copy(
                    src_ref=kvr.at[kv, hh], dst_ref=kvr.at[kv, hh],
                    send_sem=ysend_sems.at[kv, hh],
                    recv_sem=recv_sems.at[kv, hh],
                    device_id=ypeer, device_id_type=_MESH,
                ).start()

        @pl.when(pid % 2 != my_y)
        def _():
            for kv in (0, 1):
                pltpu.make_async_remote_copy(
                    src_ref=kvr.at[kv, pid], dst_ref=kvr.at[kv, pid],
                    send_sem=xsend_sems.at[kv, pid],
                    recv_sem=recv_sems.at[kv, pid],
                    device_id=xpeer, device_id_type=_MESH,
                ).wait_recv()

        q = q_ref[...]
        s1 = lax.dot_general(q, k_ref[...], dn_qk,
                             preferred_element_type=jnp.float32) * scale
        s2 = lax.dot_general(q, kvr[0, pid], dn_qk,
                             preferred_element_type=jnp.float32) * scale
        p1 = jnp.exp(s1)
        p2 = jnp.exp(s2)
        denom = (jnp.sum(p1, axis=1, keepdims=True)
                 + jnp.sum(p2, axis=1, keepdims=True))
        o = lax.dot_general(p1, v_ref[...], dn_pv,
                            preferred_element_type=jnp.float32)
        o = o + lax.dot_general(p2, kvr[1, pid], dn_pv,
                                preferred_element_type=jnp.float32)
        out_ref[...] = o / denom

        @pl.when(pid == h - 1)
        def _():
            for j in range(h // 2):
                hh = 2 * j + my_y
                for kv in (0, 1):
                    for sems, peer in ((xsend_sems, xpeer),
                                       (ysend_sems, ypeer)):
                        pltpu.make_async_remote_copy(
                            src_ref=kvr.at[kv, hh], dst_ref=kvr.at[kv, hh],
                            send_sem=sems.at[kv, hh],
                            recv_sem=recv_sems.at[kv, hh],
                            device_id=peer, device_id_type=_MESH,
                        ).wait_send()

    col = pl.BlockSpec((s, d), lambda i: (0, i))
    return pl.pallas_call(
        body,
        grid=(h,),
        out_shape=jax.ShapeDtypeStruct((s, h * d), jnp.float32),
        in_specs=[
            col, col, col,
            pl.BlockSpec(memory_space=pl.ANY),
            pl.BlockSpec(memory_space=pl.ANY),
        ],
        out_specs=col,
        scratch_shapes=[
            pltpu.VMEM((2, h, s, d), jnp.float32),
            pltpu.SemaphoreType.DMA((2, h)),
            pltpu.SemaphoreType.DMA((2, h)),
            pltpu.SemaphoreType.DMA((2, h)),
        ],
        compiler_params=pltpu.CompilerParams(collective_id=0),
    )(Q2, K2, V2, K2, V2)


def kernel(Q, K, V):
    b, s, h, d = Q.shape

    Q2 = Q.reshape(s, h * d)
    K2 = K.reshape(s, h * d)
    V2 = V.reshape(s, h * d)

    out2 = _fused(Q2, K2, V2, s, h, d)
    return out2.reshape(b, s, h, d)


# device time: 123751 ns/iter; 2.7056x vs baseline; 1.2837x over previous
import jax
import jax.numpy as jnp
from jax import lax
from jax.experimental import pallas as pl
from jax.experimental.pallas import tpu as pltpu

_MESH = pl.DeviceIdType.MESH


def _headmajor_bf16(K2, V2, s, h, d):

    def body(k_ref, v_ref, kh_ref, vh_ref):
        kh_ref[0] = k_ref[...].astype(jnp.bfloat16)
        vh_ref[0] = v_ref[...].astype(jnp.bfloat16)

    col = pl.BlockSpec((s, d), lambda i: (0, i))
    head = pl.BlockSpec((1, s, d), lambda i: (i, 0, 0))
    return pl.pallas_call(
        body,
        grid=(h,),
        out_shape=[
            jax.ShapeDtypeStruct((h, s, d), jnp.bfloat16),
            jax.ShapeDtypeStruct((h, s, d), jnp.bfloat16),
        ],
        in_specs=[col, col],
        out_specs=[head, head],
    )(K2, V2)


def _fused(Q2, Khb, Vhb, s, h, d):
    scale = d ** -0.5
    dn_qk = (((1,), (1,)), ((), ()))
    dn_pv = (((1,), (0,)), ((), ()))

    def body(q_ref, k_ref, v_ref, kh_ref, vh_ref, out_ref,
             kvr, recv_sems, xsend_sems, ysend_sems):
        pid = pl.program_id(0)
        my_x = lax.axis_index("x")
        my_y = lax.axis_index("y")
        xpeer = (1 - my_x, my_y)
        ypeer = (my_x, 1 - my_y)

        @pl.when(pid == 0)
        def _():
            barrier_sem = pltpu.get_barrier_semaphore()
            for p in (xpeer, ypeer):
                pl.semaphore_signal(barrier_sem, inc=1, device_id=p,
                                    device_id_type=_MESH)
            pl.semaphore_wait(barrier_sem, 2)
            for j in range(h // 2):
                hh = 2 * j + my_y
                for kv, src in ((0, kh_ref), (1, vh_ref)):
                    pltpu.make_async_remote_copy(
                        src_ref=src.at[hh], dst_ref=kvr.at[kv, hh],
                        send_sem=xsend_sems.at[kv, hh],
                        recv_sem=recv_sems.at[kv, hh],
                        device_id=xpeer, device_id_type=_MESH,
                    ).start()

        @pl.when(pid < h // 2)
        def _():
            hh = 2 * pid + my_y
            for kv in (0, 1):
                pltpu.make_async_remote_copy(
                    src_ref=kvr.at[kv, hh], dst_ref=kvr.at[kv, hh],
                    send_sem=xsend_sems.at[kv, hh],
                    recv_sem=recv_sems.at[kv, hh],
                    device_id=xpeer, device_id_type=_MESH,
                ).wait_recv()
                pltpu.make_async_remote_copy(
                    src_ref=kvr.at[kv, hh], dst_ref=kvr.at[kv, hh],
                    send_sem=ysend_sems.at[kv, hh],
                    recv_sem=recv_sems.at[kv, hh],
                    device_id=ypeer, device_id_type=_MESH,
                ).start()

        @pl.when(pid % 2 != my_y)
        def _():
            for kv in (0, 1):
                pltpu.make_async_remote_copy(
                    src_ref=kvr.at[kv, pid], dst_ref=kvr.at[kv, pid],
                    send_sem=xsend_sems.at[kv, pid],
                    recv_sem=recv_sems.at[kv, pid],
                    device_id=xpeer, device_id_type=_MESH,
                ).wait_recv()

        q = q_ref[...].astype(jnp.bfloat16)
        s1 = lax.dot_general(q, k_ref[0], dn_qk,
                             preferred_element_type=jnp.float32) * scale
        s2 = lax.dot_general(q, kvr[0, pid], dn_qk,
                             preferred_element_type=jnp.float32) * scale
        p1 = jnp.exp(s1)
        p2 = jnp.exp(s2)
        denom = (jnp.sum(p1, axis=1, keepdims=True)
                 + jnp.sum(p2, axis=1, keepdims=True))
        o = lax.dot_general(p1.astype(jnp.bfloat16), v_ref[0], dn_pv,
                            preferred_element_type=jnp.float32)
        o = o + lax.dot_general(p2.astype(jnp.bfloat16), kvr[1, pid], dn_pv,
                                preferred_element_type=jnp.float32)
        out_ref[...] = o / denom

        @pl.when(pid == h - 1)
        def _():
            for j in range(h // 2):
                hh = 2 * j + my_y
                for kv in (0, 1):
                    for sems, peer in ((xsend_sems, xpeer),
                                       (ysend_sems, ypeer)):
                        pltpu.make_async_remote_copy(
                            src_ref=kvr.at[kv, hh], dst_ref=kvr.at[kv, hh],
                            send_sem=sems.at[kv, hh],
                            recv_sem=recv_sems.at[kv, hh],
                            device_id=peer, device_id_type=_MESH,
                        ).wait_send()

    col = pl.BlockSpec((s, d), lambda i: (0, i))
    head = pl.BlockSpec((1, s, d), lambda i: (i, 0, 0))
    return pl.pallas_call(
        body,
        grid=(h,),
        out_shape=jax.ShapeDtypeStruct((s, h * d), jnp.float32),
        in_specs=[
            col, head, head,
            pl.BlockSpec(memory_space=pl.ANY),
            pl.BlockSpec(memory_space=pl.ANY),
        ],
        out_specs=col,
        scratch_shapes=[
            pltpu.VMEM((2, h, s, d), jnp.bfloat16),
            pltpu.SemaphoreType.DMA((2, h)),
            pltpu.SemaphoreType.DMA((2, h)),
            pltpu.SemaphoreType.DMA((2, h)),
        ],
        compiler_params=pltpu.CompilerParams(collective_id=0),
    )(Q2, Khb, Vhb, Khb, Vhb)


def kernel(Q, K, V):
    b, s, h, d = Q.shape

    Q2 = Q.reshape(s, h * d)
    K2 = K.reshape(s, h * d)
    V2 = V.reshape(s, h * d)

    Khb, Vhb = _headmajor_bf16(K2, V2, s, h, d)
    out2 = _fused(Q2, Khb, Vhb, s, h, d)
    return out2.reshape(b, s, h, d)
